# baseline (speedup 1.0000x reference)
import functools

import jax
import jax.numpy as jnp
from jax import lax
from jax.experimental import pallas as pl
from jax.experimental.pallas import tpu as pltpu
from jax.experimental.pallas import tpu_sc as plsc

N = 10000
D = 128
E = 160000
S = 5
NC, NS = 2, 16
NW = NC * NS
NPAD = 10240
NODES_W = NPAD // NW
UNIT = 16
UNITS = NODES_W // UNIT
NBUF = 4
EPT = E // NS
EVR = EPT // 16
ZCH = NPAD // NS


def _sc_body(subtok_hbm, table_hbm, dsta_hbm, dstn_hbm,
             mean_hbm, hasa_hbm, hasn_hbm,
             idx_v, rows0, rows1, rows2, rows3,
             acc0, acc1, acc2, acc3,
             eidx_v, ones_v, zeros_v, shared_v,
             g0, g1, g2, g3, o0, o1, o2, o3, sem2):
    rows = [rows0, rows1, rows2, rows3]
    accs = [acc0, acc1, acc2, acc3]
    gsem = [g0, g1, g2, g3]
    osem = [o0, o1, o2, o3]
    c = lax.axis_index("c")
    s = lax.axis_index("s")
    w = s * NC + c

    def zv(k, ck):
        zeros_v[pl.ds(k * 16, 16)] = jnp.zeros((16,), jnp.float32)
        return ck

    lax.fori_loop(0, ZCH // 16, zv, 0)
    ones_v[...] = jnp.ones((16,), jnp.float32)
    pltpu.sync_copy(zeros_v, shared_v.at[pl.ds(s * ZCH, ZCH)])

    @pl.when(c == 0)
    def _():
        pltpu.sync_copy(dsta_hbm.at[s], eidx_v)

    @pl.when(c == 1)
    def _():
        pltpu.sync_copy(dstn_hbm.at[s], eidx_v)

    plsc.subcore_barrier()

    def smask(j, cj):
        ev = eidx_v[pl.ds(j * 16, 16)]
        pltpu.async_copy(ones_v, shared_v.at[ev], sem2, add=True)
        return cj

    lax.fori_loop(0, EVR, smask, 0)

    pltpu.sync_copy(subtok_hbm.at[w], idx_v)

    def fire(u, b):
        for j in range(S):
            tok = idx_v[u, pl.ds(j * 16, 16)]
            pltpu.async_copy(table_hbm.at[tok],
                             rows[b].at[pl.ds(j * 16, 16)], gsem[b])

    for b in range(NBUF):
        fire(b, b)

    def drain_gather(b):
        for j in range(S):
            pltpu.make_async_copy(table_hbm.at[idx_v[0, pl.ds(0, 16)]],
                                  rows[b].at[pl.ds(0, 16)], gsem[b]).wait()

    def group_body(g5, carry):
        g = g5 * NBUF
        for b in range(NBUF):
            u = g + b
            drain_gather(b)
            base = w * NODES_W + u * UNIT

            @pl.when(g5 > 0)
            def _():
                pltpu.make_async_copy(accs[b], mean_hbm.at[pl.ds(0, UNIT)],
                                      osem[b]).wait()

            r = rows[b]
            acc = accs[b]

            def node_body(n, _n):
                m = 5 * n
                for v in range(D // 16):
                    col = pl.ds(v * 16, 16)
                    acc[n, col] = (r[m, col] + r[m + 1, col] + r[m + 2, col]
                                   + r[m + 3, col] + r[m + 4, col])
                return _n

            lax.fori_loop(0, UNIT, node_body, 0)

            pltpu.async_copy(accs[b], mean_hbm.at[pl.ds(base, UNIT)], osem[b])
            un = u + NBUF

            @pl.when(un < UNITS)
            def _():
                fire(un, b)
        return carry

    lax.fori_loop(0, UNITS // NBUF, group_body, 0)

    for b in range(NBUF):
        pltpu.make_async_copy(accs[b], mean_hbm.at[pl.ds(0, UNIT)],
                              osem[b]).wait()

    def drain_mask(j, cj):
        pltpu.make_async_copy(ones_v, shared_v.at[pl.ds(0, 16)], sem2).wait()
        return cj

    lax.fori_loop(0, EVR, drain_mask, 0)
    plsc.subcore_barrier()
    out_slice = pl.ds(s * ZCH, ZCH)

    @pl.when(c == 0)
    def _():
        pltpu.sync_copy(shared_v.at[out_slice], hasa_hbm.at[out_slice])

    @pl.when(c == 1)
    def _():
        pltpu.sync_copy(shared_v.at[out_slice], hasn_hbm.at[out_slice])


@functools.cache
def _make_sc_call():
    return pl.kernel(
        _sc_body,
        out_type=(
            jax.ShapeDtypeStruct((NPAD, D), jnp.float32),
            jax.ShapeDtypeStruct((NPAD,), jnp.float32),
            jax.ShapeDtypeStruct((NPAD,), jnp.float32),
        ),
        mesh=plsc.VectorSubcoreMesh(core_axis_name="c", subcore_axis_name="s",
                                    num_cores=NC, num_subcores=NS),
        scratch_types=(
            [pltpu.VMEM((UNITS, S * UNIT), jnp.int32)]
            + [pltpu.VMEM((S * UNIT, D), jnp.float32) for _ in range(NBUF)]
            + [pltpu.VMEM((UNIT, D), jnp.float32) for _ in range(NBUF)]
            + [
                pltpu.VMEM((EPT,), jnp.int32),
                pltpu.VMEM((16,), jnp.float32),
                pltpu.VMEM((ZCH,), jnp.float32),
                pltpu.VMEM_SHARED((NPAD,), jnp.float32),
            ]
            + [pltpu.SemaphoreType.DMA for _ in range(2 * NBUF + 1)]
        ),
    )


def _tc_body(sum_ref, sub_ref, ha_ref, hn_ref, wnode_ref, wa1_ref, wn1_ref,
             wa2_ref, wn2_ref, wih1_ref, whh1_ref, wih2_ref, whh2_ref,
             bih1_ref, bhh1_ref, bih2_ref, bhh2_ref, out_ref):
    f32 = jnp.float32

    def dot(x, wt):
        return lax.dot_general(x, wt, (((1,), (1,)), ((), ())),
                               preferred_element_type=f32)

    cnt = jnp.sum((sub_ref[...] > 0).astype(f32), axis=1, keepdims=True)
    mean = sum_ref[...] / jnp.maximum(cnt, 1.0)
    h0 = dot(mean, wnode_ref[...])
    ha = ha_ref[...] > 0.0
    hn = hn_ref[...] > 0.0

    def gf_of(state, wa, wn):
        a = jnp.where(ha, dot(state, wa), 0.0)
        b = jnp.where(hn, dot(state, wn), 0.0)
        return jnp.maximum(a, b)

    def gru_steps(gi, h, whh, bhh):
        i_r, i_z, i_n = gi[:, :128], gi[:, 128:256], gi[:, 256:]
        for _ in range(2):
            gh = dot(h, whh) + bhh
            r = jax.nn.sigmoid(i_r + gh[:, :128])
            z = jax.nn.sigmoid(i_z + gh[:, 128:256])
            nn_ = jnp.tanh(i_n + r * gh[:, 256:])
            h = (1.0 - z) * nn_ + z * h
        return h

    gf1 = gf_of(h0, wa1_ref[...], wn1_ref[...])
    gi1 = dot(gf1, wih1_ref[...]) + bih1_ref[...]
    h1 = gru_steps(gi1, h0, whh1_ref[...], bhh1_ref[...])

    gf2 = gf_of(h1, wa2_ref[...], wn2_ref[...])
    gi2 = (dot(h0, wih2_ref[:, :128]) + dot(gf2, wih2_ref[:, 128:])
           + bih2_ref[...])
    out_ref[...] = gru_steps(gi2, h1, whh2_ref[...], bhh2_ref[...])


BN = 400


def _make_tc_call(interpret=False):
    blk = lambda i: (i, 0)
    fix = lambda i: (0, 0)
    return pl.pallas_call(
        _tc_body,
        grid=(N // BN,),
        in_specs=[
            pl.BlockSpec((BN, D), blk),
            pl.BlockSpec((BN, 8), blk),
            pl.BlockSpec((BN, 1), blk),
            pl.BlockSpec((BN, 1), blk),
            pl.BlockSpec((D, D), fix),
            pl.BlockSpec((D, D), fix),
            pl.BlockSpec((D, D), fix),
            pl.BlockSpec((D, D), fix),
            pl.BlockSpec((D, D), fix),
            pl.BlockSpec((3 * D, D), fix),
            pl.BlockSpec((3 * D, D), fix),
            pl.BlockSpec((3 * D, 2 * D), fix),
            pl.BlockSpec((3 * D, D), fix),
            pl.BlockSpec((1, 3 * D), fix),
            pl.BlockSpec((1, 3 * D), fix),
            pl.BlockSpec((1, 3 * D), fix),
            pl.BlockSpec((1, 3 * D), fix),
        ],
        out_specs=pl.BlockSpec((BN, D), blk),
        out_shape=jax.ShapeDtypeStruct((N, D), jnp.float32),
        interpret=interpret,
    )


_tc_call = _make_tc_call()


def kernel(subtokens, edge_index_ast, edge_index_ncs, emb_table, W_node,
           W_ast1, W_ncs1, W_ast2, W_ncs2,
           w_ih1, w_hh1, b_ih1, b_hh1,
           w_ih2, w_hh2, b_ih2, b_hh2):
    sub = subtokens.astype(jnp.int32)
    sub_p = jnp.pad(sub, ((0, NPAD - N), (0, 0))).reshape(NW, UNITS, S * UNIT)
    sub8 = jnp.pad(sub, ((0, 0), (0, 8 - S)))
    dsta = edge_index_ast[1].astype(jnp.int32).reshape(NS, EPT)
    dstn = edge_index_ncs[1].astype(jnp.int32).reshape(NS, EPT)

    esum, hasa, hasn = _make_sc_call()(sub_p, emb_table, dsta, dstn)

    return _tc_call(esum, sub8,
                    hasa.reshape(NPAD, 1), hasn.reshape(NPAD, 1),
                    W_node, W_ast1, W_ncs1, W_ast2, W_ncs2,
                    w_ih1, w_hh1, w_ih2, w_hh2,
                    b_ih1.reshape(1, -1), b_hh1.reshape(1, -1),
                    b_ih2.reshape(1, -1), b_hh2.reshape(1, -1))
